# Initial kernel scaffold; baseline (speedup 1.0000x reference)
import jax
import jax.numpy as jnp
from jax.experimental import pallas as pl


def kernel(z_n, source_idx, target_idx, W_enc, W_dec, c, d):
    raise NotImplementedError("write your pallas kernel here")


# speedup vs baseline: 1.9121x; 1.9121x over previous
import functools

import jax
import jax.numpy as jnp
from jax import lax
from jax.experimental import pallas as pl
from jax.experimental.pallas import tpu as pltpu

NUM_CHARTS = 8
LATENT_DIM = 1024
RANK = 512
B = 2048
BLK = 256


def _fused_body(z_ref, src_ref, tgt_ref, wenc_ref, wdec_ref, c_ref, d_ref, out_ref):
    zb = z_ref[...]
    src = src_ref[...]
    tgt = tgt_ref[...]

    h = jnp.zeros((BLK, RANK), jnp.float32)
    for e in range(NUM_CHARTS):
        he = lax.dot_general(zb, wenc_ref[e], (((1,), (1,)), ((), ())),
                             preferred_element_type=jnp.float32)
        h = jnp.where(src == e, he, h)
    eye = lax.broadcasted_iota(jnp.int32, (BLK, NUM_CHARTS), 1)
    onehot_s = (src == eye).astype(jnp.bfloat16)
    h = h + lax.dot_general(onehot_s, c_ref[...], (((1,), (0,)), ((), ())),
                            preferred_element_type=jnp.float32)

    hb = h.astype(jnp.bfloat16)
    y = jnp.zeros((BLK, LATENT_DIM), jnp.float32)
    for e in range(NUM_CHARTS):
        ye = lax.dot_general(hb, wdec_ref[e], (((1,), (1,)), ((), ())),
                             preferred_element_type=jnp.float32)
        y = jnp.where(tgt == e, ye, y)
    onehot_t = (tgt == eye).astype(jnp.bfloat16)
    y = y + lax.dot_general(onehot_t, d_ref[...], (((1,), (0,)), ((), ())),
                            preferred_element_type=jnp.float32)
    out_ref[...] = y


@jax.jit
def kernel(z_n, source_idx, target_idx, W_enc, W_dec, c, d):
    zb = z_n.astype(jnp.bfloat16)
    wenc = W_enc.astype(jnp.bfloat16)
    wdec = W_dec.astype(jnp.bfloat16)
    cb = c.astype(jnp.bfloat16)
    db = d.astype(jnp.bfloat16)
    src = source_idx.astype(jnp.int32).reshape(B, 1)
    tgt = target_idx.astype(jnp.int32).reshape(B, 1)

    grid = (B // BLK,)
    out = pl.pallas_call(
        _fused_body,
        grid=grid,
        in_specs=[
            pl.BlockSpec((BLK, LATENT_DIM), lambda i: (i, 0)),
            pl.BlockSpec((BLK, 1), lambda i: (i, 0)),
            pl.BlockSpec((BLK, 1), lambda i: (i, 0)),
            pl.BlockSpec((NUM_CHARTS, RANK, LATENT_DIM), lambda i: (0, 0, 0)),
            pl.BlockSpec((NUM_CHARTS, LATENT_DIM, RANK), lambda i: (0, 0, 0)),
            pl.BlockSpec((NUM_CHARTS, RANK), lambda i: (0, 0)),
            pl.BlockSpec((NUM_CHARTS, LATENT_DIM), lambda i: (0, 0)),
        ],
        out_specs=pl.BlockSpec((BLK, LATENT_DIM), lambda i: (i, 0)),
        out_shape=jax.ShapeDtypeStruct((B, LATENT_DIM), jnp.float32),
    )(zb, src, tgt, wenc, wdec, cb, db)
    return out
